# Initial kernel scaffold; baseline (speedup 1.0000x reference)
import jax
import jax.numpy as jnp
from jax.experimental import pallas as pl


def kernel(hidden_states, weight):
    raise NotImplementedError("write your pallas kernel here")


# speedup vs baseline: 1.1329x; 1.1329x over previous
import jax
import jax.numpy as jnp
from jax.experimental import pallas as pl
from jax.experimental.pallas import tpu as pltpu

_E = 64
_K = 8
_TBLK = 512


def _router_block(x_ref, w_ref, probs_ref, vals_ref, idx_ref):
    x = x_ref[...]
    w = w_ref[...]
    logits = jax.lax.dot_general(
        x, w, (((1,), (1,)), ((), ())),
        preferred_element_type=jnp.float32,
    )
    m = jnp.max(logits, axis=-1, keepdims=True)
    e = jnp.exp(logits - m)
    denom = jnp.sum(e, axis=-1, keepdims=True)
    probs = e / denom
    probs_ref[...] = probs

    t = probs.shape[0]
    lane = jax.lax.broadcasted_iota(jnp.int32, (t, _E), 1)
    work = probs
    vals = []
    idxs = []
    for _ in range(_K):
        mx = jnp.max(work, axis=-1, keepdims=True)
        ix = jnp.min(jnp.where(work == mx, lane, _E), axis=-1,
                     keepdims=True)
        vals.append(mx)
        idxs.append(ix)
        work = jnp.where(lane == ix, -1.0, work)
    top_vals = jnp.concatenate(vals, axis=-1)
    top_idx = jnp.concatenate(idxs, axis=-1)
    vals_ref[...] = top_vals / jnp.sum(top_vals, axis=-1, keepdims=True)
    idx_ref[...] = top_idx


def kernel(hidden_states, weight):
    b, s, h = hidden_states.shape
    e = weight.shape[0]
    n = b * s
    x = hidden_states.reshape(n, h)
    grid = (n // _TBLK,)
    probs, top_vals, top_idx = pl.pallas_call(
        _router_block,
        grid=grid,
        in_specs=[
            pl.BlockSpec((_TBLK, h), lambda i: (i, 0)),
            pl.BlockSpec((e, h), lambda i: (0, 0)),
        ],
        out_specs=[
            pl.BlockSpec((_TBLK, e), lambda i: (i, 0)),
            pl.BlockSpec((_TBLK, _K), lambda i: (i, 0)),
            pl.BlockSpec((_TBLK, _K), lambda i: (i, 0)),
        ],
        out_shape=[
            jax.ShapeDtypeStruct((n, e), jnp.float32),
            jax.ShapeDtypeStruct((n, _K), jnp.float32),
            jax.ShapeDtypeStruct((n, _K), jnp.int32),
        ],
        compiler_params=pltpu.CompilerParams(
            dimension_semantics=("parallel",),
        ),
    )(x, weight)
    return (
        probs.reshape(b, s, e),
        top_vals.reshape(b, s, _K),
        top_idx.reshape(b, s, _K),
    )


# speedup vs baseline: 1.2854x; 1.1346x over previous
import jax
import jax.numpy as jnp
from jax.experimental import pallas as pl
from jax.experimental.pallas import tpu as pltpu

_E = 64
_K = 8
_TBLK = 1024


def _router_block(x_ref, w_ref, probs_ref, vals_ref, idx_ref):
    x = x_ref[...]
    w = w_ref[...]
    logits = jax.lax.dot_general(
        x, w, (((1,), (1,)), ((), ())),
        preferred_element_type=jnp.float32,
    )
    m = jnp.max(logits, axis=-1, keepdims=True)
    e = jnp.exp(logits - m)
    denom = jnp.sum(e, axis=-1, keepdims=True)
    probs = e / denom
    probs_ref[...] = probs

    t = probs.shape[0]
    lane = jax.lax.broadcasted_iota(jnp.int32, (t, _E), 1)
    work = probs
    vals = []
    idxs = []
    for _ in range(_K):
        mx = jnp.max(work, axis=-1, keepdims=True)
        ix = jnp.min(jnp.where(work == mx, lane, _E), axis=-1,
                     keepdims=True)
        vals.append(mx)
        idxs.append(ix)
        work = jnp.where(lane == ix, -1.0, work)
    top_vals = jnp.concatenate(vals, axis=-1)
    top_idx = jnp.concatenate(idxs, axis=-1)
    vals_ref[...] = top_vals / jnp.sum(top_vals, axis=-1, keepdims=True)
    idx_ref[...] = top_idx


def kernel(hidden_states, weight):
    b, s, h = hidden_states.shape
    e = weight.shape[0]
    n = b * s
    x = hidden_states.reshape(n, h)
    grid = (n // _TBLK,)
    probs, top_vals, top_idx = pl.pallas_call(
        _router_block,
        grid=grid,
        in_specs=[
            pl.BlockSpec((_TBLK, h), lambda i: (i, 0)),
            pl.BlockSpec((e, h), lambda i: (0, 0)),
        ],
        out_specs=[
            pl.BlockSpec((_TBLK, e), lambda i: (i, 0)),
            pl.BlockSpec((_TBLK, _K), lambda i: (i, 0)),
            pl.BlockSpec((_TBLK, _K), lambda i: (i, 0)),
        ],
        out_shape=[
            jax.ShapeDtypeStruct((n, e), jnp.float32),
            jax.ShapeDtypeStruct((n, _K), jnp.float32),
            jax.ShapeDtypeStruct((n, _K), jnp.int32),
        ],
        compiler_params=pltpu.CompilerParams(
            dimension_semantics=("parallel",),
        ),
    )(x, weight)
    return (
        probs.reshape(b, s, e),
        top_vals.reshape(b, s, _K),
        top_idx.reshape(b, s, _K),
    )
